# baseline (speedup 1.0000x reference)
import jax
import jax.numpy as jnp
from jax import lax
from jax.experimental import pallas as pl
from jax.experimental.pallas import tpu as pltpu

_F32 = jnp.float32
_PREC = lax.Precision.DEFAULT
_DN_TRANS_B = (((1,), (1,)), ((), ()))


def _fused_kernel(xt_ref, xf_ref, we_ref, be_ref, wi_ref, bw_ref,
                  o_ref, w_ref, *, tb, e_dim, ci):
    @pl.when(pl.program_id(0) == 0)
    def _init_w():
        w_ref[...] = lax.dot_general(
            xf_ref[...], wi_ref[...], _DN_TRANS_B,
            preferred_element_type=_F32, precision=_PREC) + bw_ref[...]

    emb = lax.dot_general(xt_ref[...], we_ref[...], _DN_TRANS_B,
                          preferred_element_type=_F32,
                          precision=_PREC) + be_ref[...]
    for i0 in range(0, e_dim, ci):
        wc = w_ref[i0:i0 + ci, :]
        for b in range(tb):
            ej = emb[b:b + 1, :]
            ei = jnp.transpose(emb[b:b + 1, i0:i0 + ci])
            o_ref[b, i0:i0 + ci, :] = ei * (ej * wc)


def kernel(x, w_embed, b_embed, w_inter, b_inter):
    B, D = x.shape
    E = w_embed.shape[0]
    assert B == E, "interaction mechanism requires batch_size == embed_dim"
    xh = x.astype(jnp.bfloat16)
    we = w_embed.astype(jnp.bfloat16)
    wi = w_inter.astype(jnp.bfloat16)
    be = b_embed.astype(_F32).reshape(1, E)
    bw = b_inter.astype(_F32).reshape(1, E)

    tb = 8 if B % 8 == 0 else B
    nb = B // tb
    ci = 128 if E % 128 == 0 else E
    cparams = pltpu.CompilerParams(
        dimension_semantics=("arbitrary",),
        vmem_limit_bytes=61 << 20)
    return pl.pallas_call(
        lambda *refs: _fused_kernel(*refs, tb=tb, e_dim=E, ci=ci),
        out_shape=jax.ShapeDtypeStruct((B, E, E), _F32),
        grid=(nb,),
        in_specs=[
            pl.BlockSpec((tb, D), lambda b: (b, 0)),
            pl.BlockSpec((B, D), lambda b: (0, 0)),
            pl.BlockSpec((E, D), lambda b: (0, 0)),
            pl.BlockSpec((1, E), lambda b: (0, 0)),
            pl.BlockSpec((E, D), lambda b: (0, 0)),
            pl.BlockSpec((1, E), lambda b: (0, 0)),
        ],
        out_specs=pl.BlockSpec((tb, E, E), lambda b: (b, 0, 0)),
        scratch_shapes=[pltpu.VMEM((E, E), _F32)],
        compiler_params=cparams,
    )(xh, xh, we, be, wi, bw)


# speedup vs baseline: 1.0045x; 1.0045x over previous
import jax
import jax.numpy as jnp
from jax import lax
from jax.experimental import pallas as pl
from jax.experimental.pallas import tpu as pltpu

_F32 = jnp.float32
_PREC = lax.Precision.DEFAULT


_DN_TRANS_B = (((1,), (1,)), ((), ()))


def _proj_kernel(x_ref, we_ref, be_ref, ww_ref, bw_ref,
                 emb_ref, w_ref):
    x = x_ref[...]
    emb_ref[...] = lax.dot_general(x, we_ref[...], _DN_TRANS_B,
                                   preferred_element_type=_F32,
                                   precision=_PREC) + be_ref[...]
    w_ref[...] = lax.dot_general(x, ww_ref[...], _DN_TRANS_B,
                                 preferred_element_type=_F32,
                                 precision=_PREC) + bw_ref[...]


def _interact_kernel(emb_ref, w_ref, o_ref, *, tb, e_dim, ci):
    for i0 in range(0, e_dim, ci):
        wc = w_ref[i0:i0 + ci, :]
        for b in range(tb):
            ej = emb_ref[b:b + 1, :]
            ei = jnp.transpose(emb_ref[b:b + 1, i0:i0 + ci])
            o_ref[b, i0:i0 + ci, :] = ei * (ej * wc)


def _project(x, we, be, ww, bw):
    B, D = x.shape
    E = we.shape[0]
    nc = 2 if E % 256 == 0 else 1
    ec = E // nc
    cparams = pltpu.CompilerParams(
        dimension_semantics=("parallel",),
        vmem_limit_bytes=56 << 20)
    return pl.pallas_call(
        _proj_kernel,
        out_shape=(jax.ShapeDtypeStruct((B, E), _F32),
                   jax.ShapeDtypeStruct((B, E), _F32)),
        grid=(nc,),
        in_specs=[
            pl.BlockSpec((B, D), lambda c: (0, 0)),
            pl.BlockSpec((ec, D), lambda c: (c, 0)),
            pl.BlockSpec((1, ec), lambda c: (0, c)),
            pl.BlockSpec((ec, D), lambda c: (c, 0)),
            pl.BlockSpec((1, ec), lambda c: (0, c)),
        ],
        out_specs=(pl.BlockSpec((B, ec), lambda c: (0, c)),
                   pl.BlockSpec((B, ec), lambda c: (0, c))),
        compiler_params=cparams,
    )(x, we, be, ww, bw)


def _interact(emb, w):
    B, E = emb.shape
    tb = 8 if B % 8 == 0 else B
    nb = B // tb
    ci = 128 if E % 128 == 0 else E
    out_block = tb * E * E * 4
    cparams = pltpu.CompilerParams(
        dimension_semantics=("parallel",),
        vmem_limit_bytes=int(min(60 << 20, 2 * out_block + (8 << 20))))
    return pl.pallas_call(
        lambda er, wr, orf: _interact_kernel(er, wr, orf,
                                             tb=tb, e_dim=E, ci=ci),
        out_shape=jax.ShapeDtypeStruct((B, E, E), _F32),
        grid=(nb,),
        in_specs=[
            pl.BlockSpec((tb, E), lambda b: (b, 0)),
            pl.BlockSpec((E, E), lambda b: (0, 0)),
        ],
        out_specs=pl.BlockSpec((tb, E, E), lambda b: (b, 0, 0)),
        compiler_params=cparams,
    )(emb, w)


def kernel(x, w_embed, b_embed, w_inter, b_inter):
    B, D = x.shape
    E = w_embed.shape[0]
    assert B == E, "interaction mechanism requires batch_size == embed_dim"
    x = x.astype(_F32)
    be = b_embed.astype(_F32).reshape(1, E)
    bw = b_inter.astype(_F32).reshape(1, E)
    emb, w = _project(x, w_embed.astype(_F32), be, w_inter.astype(_F32), bw)
    return _interact(emb, w)


# speedup vs baseline: 1.6004x; 1.5932x over previous
import jax
import jax.numpy as jnp
from jax import lax
from jax.experimental import pallas as pl
from jax.experimental.pallas import tpu as pltpu
from jax.sharding import Mesh, PartitionSpec as P

_F32 = jnp.float32
_PREC = lax.Precision.DEFAULT
_DN_TRANS_B = (((1,), (1,)), ((), ()))


def _proj_kernel(xloc_ref, xf_ref, we_ref, be_ref, wi_ref, bw_ref,
                 emb_ref, w_ref):
    emb_ref[...] = lax.dot_general(xloc_ref[...], we_ref[...], _DN_TRANS_B,
                                   preferred_element_type=_F32,
                                   precision=_PREC) + be_ref[...]
    w_ref[...] = lax.dot_general(xf_ref[...], wi_ref[...], _DN_TRANS_B,
                                 preferred_element_type=_F32,
                                 precision=_PREC) + bw_ref[...]


def _interact_kernel(emb_ref, w_ref, o_ref, *, tb, e_dim, ci):
    for i0 in range(0, e_dim, ci):
        wc = w_ref[i0:i0 + ci, :]
        for b in range(tb):
            ej = emb_ref[b:b + 1, :]
            ei = jnp.transpose(emb_ref[b:b + 1, i0:i0 + ci])
            o_ref[b, i0:i0 + ci, :] = ei * (ej * wc)


def _run_local(xloc, xfull, we, be, wi, bw):
    hb, D = xloc.shape
    B = xfull.shape[0]
    E = we.shape[0]
    emb, w = pl.pallas_call(
        _proj_kernel,
        out_shape=(jax.ShapeDtypeStruct((hb, E), _F32),
                   jax.ShapeDtypeStruct((B, E), _F32)),
        compiler_params=pltpu.CompilerParams(vmem_limit_bytes=48 << 20),
    )(xloc, xfull, we, be, wi, bw)

    tb = 8 if hb % 8 == 0 else hb
    nb = hb // tb
    ci = 128 if E % 128 == 0 else E
    out_block = tb * E * E * 4
    cparams = pltpu.CompilerParams(
        dimension_semantics=("arbitrary",),
        vmem_limit_bytes=int(min(60 << 20, 2 * out_block + (8 << 20))))
    return pl.pallas_call(
        lambda er, wr, orf: _interact_kernel(er, wr, orf,
                                             tb=tb, e_dim=E, ci=ci),
        out_shape=jax.ShapeDtypeStruct((hb, E, E), _F32),
        grid=(nb,),
        in_specs=[
            pl.BlockSpec((tb, E), lambda b: (b, 0)),
            pl.BlockSpec((E, E), lambda b: (0, 0)),
        ],
        out_specs=pl.BlockSpec((tb, E, E), lambda b: (b, 0, 0)),
        compiler_params=cparams,
    )(emb, w)


def kernel(x, w_embed, b_embed, w_inter, b_inter):
    B, D = x.shape
    E = w_embed.shape[0]
    assert B == E, "interaction mechanism requires batch_size == embed_dim"
    xh = x.astype(_F32)
    we = w_embed.astype(_F32)
    wi = w_inter.astype(_F32)
    be = b_embed.astype(_F32).reshape(1, E)
    bw = b_inter.astype(_F32).reshape(1, E)

    devs = [d for d in jax.devices() if d.platform == "tpu"][:2]
    if len(devs) == 2 and B % 16 == 0:
        hb = B // 2
        mesh = Mesh(devs, ("d",))

        def _sharded(xh, we, be, wi, bw):
            idx = lax.axis_index("d")
            xloc = lax.dynamic_slice_in_dim(xh, idx * hb, hb, 0)
            return _run_local(xloc, xh, we, be, wi, bw)

        rep = P()
        return jax.shard_map(
            _sharded, mesh=mesh,
            in_specs=(rep, rep, rep, rep, rep),
            out_specs=P("d", None, None),
            check_vma=False,
        )(xh, we, be, wi, bw)
    return _run_local(xh, xh, we, be, wi, bw)
